# baseline (speedup 1.0000x reference)
import jax
import jax.numpy as jnp
from jax import lax
from jax.experimental import pallas as pl

_S3 = 8
_C = 128


def _feat_body(f_ref, o_ref):
    b = f_ref.shape[0]
    rep = jnp.broadcast_to(f_ref[...][:, None, :], (b, _S3, _C))
    o_ref[...] = rep.reshape(b * _S3, _C)


def _coord_body(c_ref, o_ref):
    b = c_ref.shape[0]
    jj = lax.broadcasted_iota(jnp.int32, (b, _S3, 3), 1)
    kk = lax.broadcasted_iota(jnp.int32, (b, _S3, 3), 2)
    off = lax.shift_right_logical(jj, 2 - kk) & 1
    fine = c_ref[...][:, None, :] * 2 + off
    o_ref[...] = fine.reshape(b * _S3, 3)


def kernel(features, coords):
    n, c = features.shape
    bf = 1000
    fine_feats = pl.pallas_call(
        _feat_body,
        grid=(n // bf,),
        in_specs=[pl.BlockSpec((bf, c), lambda i: (i, 0))],
        out_specs=pl.BlockSpec((bf * _S3, c), lambda i: (i, 0)),
        out_shape=jax.ShapeDtypeStruct((n * _S3, c), jnp.float32),
    )(features)

    bc = 2000
    fine_coords = pl.pallas_call(
        _coord_body,
        grid=(n // bc,),
        in_specs=[pl.BlockSpec((bc, 3), lambda i: (i, 0))],
        out_specs=pl.BlockSpec((bc * _S3, 3), lambda i: (i, 0)),
        out_shape=jax.ShapeDtypeStruct((n * _S3, 3), jnp.int32),
    )(coords)

    return fine_feats, fine_coords


# speedup vs baseline: 1.4799x; 1.4799x over previous
import jax
import jax.numpy as jnp
from jax import lax
from jax.experimental import pallas as pl
from jax.experimental.pallas import tpu as pltpu

_S3 = 8
_C = 128


def _feat_body(f_ref, o_ref, sem):
    i = pl.program_id(0)
    b = f_ref.shape[0]
    cps = [
        pltpu.make_async_copy(f_ref, o_ref.at[pl.ds(i * b, b), j], sem)
        for j in range(_S3)
    ]
    for c in cps:
        c.start()
    for c in cps:
        c.wait()


def _coord_body(c_ref, o_ref):
    b = c_ref.shape[1]
    rep = jnp.broadcast_to(c_ref[...][:, :, None], (3, b, _S3)).reshape(3, b * _S3)
    jj = lax.broadcasted_iota(jnp.int32, (3, b * _S3), 1) % _S3
    kk = lax.broadcasted_iota(jnp.int32, (3, b * _S3), 0)
    off = lax.shift_right_logical(jj, 2 - kk) & 1
    o_ref[...] = rep * 2 + off


def kernel(features, coords):
    n, c = features.shape
    bf = 1000
    fine3 = pl.pallas_call(
        _feat_body,
        grid=(n // bf,),
        in_specs=[pl.BlockSpec((bf, c), lambda i: (i, 0))],
        out_specs=pl.BlockSpec(memory_space=pl.ANY),
        out_shape=jax.ShapeDtypeStruct((n, _S3, c), jnp.float32),
        scratch_shapes=[pltpu.SemaphoreType.DMA],
    )(features)

    bc = 4096
    gc = -(-n // bc)
    fine_t = pl.pallas_call(
        _coord_body,
        grid=(gc,),
        in_specs=[pl.BlockSpec((3, bc), lambda i: (0, i))],
        out_specs=pl.BlockSpec((3, bc * _S3), lambda i: (0, i)),
        out_shape=jax.ShapeDtypeStruct((3, n * _S3), jnp.int32),
    )(coords.T)

    return fine3.reshape(n * _S3, c), fine_t.T


# speedup vs baseline: 1.6503x; 1.1151x over previous
import jax
import jax.numpy as jnp
from jax import lax
from jax.experimental import pallas as pl
from jax.experimental.pallas import tpu as pltpu

_S3 = 8
_C = 128


def _feat_body(f_ref, o_ref, sem):
    i = pl.program_id(0)
    b = f_ref.shape[0]
    cps = [
        pltpu.make_async_copy(f_ref, o_ref.at[pl.ds(i * b, b), j], sem)
        for j in range(_S3)
    ]
    for c in cps:
        c.start()
    for c in cps:
        c.wait()


def _coord_body(c_ref, o_ref):
    b = c_ref.shape[1]
    rep = jnp.broadcast_to(c_ref[...][:, :, None], (3, b, _S3)).reshape(3, b * _S3)
    jj = lax.broadcasted_iota(jnp.int32, (3, b * _S3), 1) % _S3
    kk = lax.broadcasted_iota(jnp.int32, (3, b * _S3), 0)
    off = lax.shift_right_logical(jj, 2 - kk) & 1
    o_ref[...] = rep * 2 + off


def kernel(features, coords):
    n, c = features.shape
    bf = 2000
    fine3 = pl.pallas_call(
        _feat_body,
        grid=(n // bf,),
        in_specs=[pl.BlockSpec((bf, c), lambda i: (i, 0))],
        out_specs=pl.BlockSpec(memory_space=pl.ANY),
        out_shape=jax.ShapeDtypeStruct((n, _S3, c), jnp.float32),
        scratch_shapes=[pltpu.SemaphoreType.DMA],
    )(features)

    bc = 4096
    gc = -(-n // bc)
    fine_t = pl.pallas_call(
        _coord_body,
        grid=(gc,),
        in_specs=[pl.BlockSpec((3, bc), lambda i: (0, i))],
        out_specs=pl.BlockSpec((3, bc * _S3), lambda i: (0, i)),
        out_shape=jax.ShapeDtypeStruct((3, n * _S3), jnp.int32),
    )(coords.T)

    return fine3.reshape(n * _S3, c), fine_t.T


# speedup vs baseline: 1.7513x; 1.0612x over previous
import jax
import jax.numpy as jnp
from jax import lax
from jax.experimental import pallas as pl
from jax.experimental.pallas import tpu as pltpu

_S3 = 8
_C = 128


def _feat_body(f_ref, o_ref, sem):
    i = pl.program_id(0)
    b = f_ref.shape[0]
    cps = [
        pltpu.make_async_copy(f_ref, o_ref.at[pl.ds(i * b, b), j], sem)
        for j in range(_S3)
    ]
    for c in cps:
        c.start()
    for c in cps:
        c.wait()


def _coord_body(c_ref, o_ref):
    b = c_ref.shape[1]
    rep = jnp.broadcast_to(c_ref[...][:, :, None], (3, b, _S3)).reshape(3, b * _S3)
    jj = lax.broadcasted_iota(jnp.int32, (3, b * _S3), 1) % _S3
    kk = lax.broadcasted_iota(jnp.int32, (3, b * _S3), 0)
    off = lax.shift_right_logical(jj, 2 - kk) & 1
    o_ref[...] = rep * 2 + off


def kernel(features, coords):
    n, c = features.shape
    bf = 5000
    fine3 = pl.pallas_call(
        _feat_body,
        grid=(n // bf,),
        in_specs=[pl.BlockSpec((bf, c), lambda i: (i, 0))],
        out_specs=pl.BlockSpec(memory_space=pl.ANY),
        out_shape=jax.ShapeDtypeStruct((n, _S3, c), jnp.float32),
        scratch_shapes=[pltpu.SemaphoreType.DMA],
    )(features)

    bc = 4096
    gc = -(-n // bc)
    fine_t = pl.pallas_call(
        _coord_body,
        grid=(gc,),
        in_specs=[pl.BlockSpec((3, bc), lambda i: (0, i))],
        out_specs=pl.BlockSpec((3, bc * _S3), lambda i: (0, i)),
        out_shape=jax.ShapeDtypeStruct((3, n * _S3), jnp.int32),
    )(coords.T)

    return fine3.reshape(n * _S3, c), fine_t.T


# speedup vs baseline: 1.8122x; 1.0348x over previous
import jax
import jax.numpy as jnp
from jax import lax
from jax.experimental import pallas as pl
from jax.experimental.pallas import tpu as pltpu

_S3 = 8
_C = 128


def _feat_body(f_ref, o_ref, sem):
    i = pl.program_id(0)
    b = f_ref.shape[0]
    cps = [
        pltpu.make_async_copy(f_ref, o_ref.at[pl.ds(i * b, b), j], sem)
        for j in range(_S3)
    ]
    for c in cps:
        c.start()
    for c in cps:
        c.wait()


def _coord_body(c_ref, o_ref):
    b = c_ref.shape[1]
    rep = jnp.broadcast_to(c_ref[...][:, :, None], (3, b, _S3)).reshape(3, b * _S3)
    jj = lax.broadcasted_iota(jnp.int32, (3, b * _S3), 1) % _S3
    kk = lax.broadcasted_iota(jnp.int32, (3, b * _S3), 0)
    off = lax.shift_right_logical(jj, 2 - kk) & 1
    o_ref[...] = rep * 2 + off


def kernel(features, coords):
    n, c = features.shape
    bf = 10000
    fine3 = pl.pallas_call(
        _feat_body,
        grid=(n // bf,),
        in_specs=[pl.BlockSpec((bf, c), lambda i: (i, 0))],
        out_specs=pl.BlockSpec(memory_space=pl.ANY),
        out_shape=jax.ShapeDtypeStruct((n, _S3, c), jnp.float32),
        scratch_shapes=[pltpu.SemaphoreType.DMA],
    )(features)

    bc = 4096
    gc = -(-n // bc)
    fine_t = pl.pallas_call(
        _coord_body,
        grid=(gc,),
        in_specs=[pl.BlockSpec((3, bc), lambda i: (0, i))],
        out_specs=pl.BlockSpec((3, bc * _S3), lambda i: (0, i)),
        out_shape=jax.ShapeDtypeStruct((3, n * _S3), jnp.int32),
    )(coords.T)

    return fine3.reshape(n * _S3, c), fine_t.T
